# baseline (speedup 1.0000x reference)
import functools

import jax
import jax.numpy as jnp
from jax import lax
from jax.experimental import pallas as pl
from jax.experimental.pallas import tpu as pltpu
from jax.experimental.pallas import tpu_sc as plsc

_N = 10000
_DEG = 32
_D = 128
_NUM = 2
_G = _N * _DEG

_SC_C = 80
_SC_K = 5
_SC_GRP = _SC_C * _SC_K
_NW = 32

_BLK = 200

_CHUNKS = ((0, 4000), (4000, 4000), (8000, 2000))


def _sc_gather(table, idx, c=_SC_C, k=_SC_K):
  grp = c * k
  g_total = idx.shape[0]
  d = table.shape[1]
  per_w = g_total // _NW
  ngrp = per_w // grp
  assert per_w % grp == 0 and ngrp % 2 == 0 and ngrp >= 4

  mesh = plsc.VectorSubcoreMesh(core_axis_name="c", subcore_axis_name="s")

  @functools.partial(
      pl.kernel,
      out_type=jax.ShapeDtypeStruct((g_total, d), jnp.int32),
      mesh=mesh,
      scratch_types=[
          pltpu.VMEM((grp,), jnp.int32),
          pltpu.VMEM((grp,), jnp.int32),
          pltpu.VMEM((grp, d), jnp.int32),
          pltpu.VMEM((grp, d), jnp.int32),
          pltpu.SemaphoreType.DMA,
          pltpu.SemaphoreType.DMA,
          pltpu.SemaphoreType.DMA,
          pltpu.SemaphoreType.DMA,
          pltpu.SemaphoreType.DMA,
      ],
  )
  def gather_k(table_hbm, idx_hbm, out_hbm, idx_v0, idx_v1, rows_v0, rows_v1,
               isem0, isem1, gsem, ssem0, ssem1):
    wid = lax.axis_index("s") * 2 + lax.axis_index("c")
    base = wid * per_w
    idx_vs = (idx_v0, idx_v1)
    rows_vs = (rows_v0, rows_v1)
    isems = (isem0, isem1)
    ssems = (ssem0, ssem1)

    del isems

    def group(g, buf, drain):
      gbase = base + g * grp
      pltpu.sync_copy(idx_hbm.at[pl.ds(gbase, grp)], idx_vs[buf])
      if drain:
        pltpu.make_async_copy(rows_vs[buf], out_hbm.at[pl.ds(base, grp)],
                              ssems[buf]).wait()
      copies = []
      for j in range(k):
        copies.append(
            pltpu.async_copy(
                table_hbm.at[idx_vs[buf].at[pl.ds(j * c, c)]],
                rows_vs[buf].at[pl.ds(j * c, c)],
                gsem,
            ))
      for cp in copies:
        cp.wait()
      pltpu.async_copy(rows_vs[buf], out_hbm.at[pl.ds(gbase, grp)],
                       ssems[buf])

    group(0, 0, False)
    group(1, 1, False)

    def pair(p, carry):
      group(2 * p, 0, True)
      group(2 * p + 1, 1, True)
      return carry

    lax.fori_loop(1, ngrp // 2, pair, 0)
    pltpu.make_async_copy(rows_v0, out_hbm.at[pl.ds(base, grp)],
                          ssem0).wait()
    pltpu.make_async_copy(rows_v1, out_hbm.at[pl.ds(base, grp)],
                          ssem1).wait()

  return gather_k(table, idx)


def _sigm(v):
  return 0.5 + 0.5 * jnp.tanh(0.5 * v)


def _unpack2(v32):
  lo = lax.bitcast_convert_type(lax.shift_left(v32, 16), jnp.float32)
  hi = lax.bitcast_convert_type(
      lax.bitwise_and(v32, jnp.int32(-65536)), jnp.float32)
  return lo, hi


def _pack2(f_lo, f_hi):
  def rne(f):
    u = lax.bitcast_convert_type(f, jnp.uint32)
    return u + jnp.uint32(0x7FFF) + (
        lax.shift_right_logical(u, jnp.uint32(16)) & jnp.uint32(1))
  lo = lax.shift_right_logical(rne(f_lo), jnp.uint32(16))
  hi = lax.bitwise_and(rne(f_hi), jnp.uint32(0xFFFF0000))
  return lax.bitcast_convert_type(lax.bitwise_or(lo, hi), jnp.int32)


def _tc_layer(hp, m, ew, w_gates, bias, w_sn, b_neigh, n0, nn, final):
  nblk = nn // _BLK
  blk0 = n0 // _BLK
  b2 = 2 * _BLK

  def body(hp_ref, m_ref, ew_ref, wg_ref, b_ref, wsn_ref, bn_ref, *out_refs):
    h0lo, h0hi = _unpack2(hp_ref[...])
    h0 = jnp.concatenate([h0lo, h0hi], axis=0)
    wg = wg_ref[...]
    b = b_ref[...]
    ht = jnp.zeros((b2, _D), jnp.float32)
    ct = jnp.zeros((b2, _D), jnp.float32)
    for t in range(_DEG):
      et = ew_ref[t]
      mlo, mhi = _unpack2(m_ref[t])
      mt = jnp.concatenate([mlo * et, mhi * et], axis=0)
      g = jnp.dot(jnp.concatenate([mt, ht], axis=1).astype(jnp.bfloat16),
                  wg, preferred_element_type=jnp.float32) + b
      ig = 0.5 + 0.5 * jnp.tanh(g[:, :_D])
      fg = 0.5 + 0.5 * jnp.tanh(g[:, _D:2 * _D])
      gg = jnp.tanh(g[:, 2 * _D:3 * _D])
      og = 0.5 + 0.5 * jnp.tanh(g[:, 3 * _D:])
      ct = fg * ct + ig * gg
      ht = og * jnp.tanh(ct)
    out = (jnp.dot(jnp.concatenate([h0, ht], axis=1), wsn_ref[...],
                   preferred_element_type=jnp.float32) + bn_ref[...])
    if final:
      out_refs[0][...] = out[:_BLK]
      out_refs[1][...] = out[_BLK:]
    else:
      out_refs[0][...] = _pack2(out[:_BLK], out[_BLK:])

  full = lambda i: (0, 0)
  if final:
    out_specs = [pl.BlockSpec((_BLK, _D), lambda i: (i, 0)),
                 pl.BlockSpec((_BLK, _D), lambda i: (i, 0))]
    out_shape = [jax.ShapeDtypeStruct((nn, _D), jnp.float32),
                 jax.ShapeDtypeStruct((nn, _D), jnp.float32)]
  else:
    out_specs = pl.BlockSpec((_BLK, _D), lambda i: (i, 0))
    out_shape = jax.ShapeDtypeStruct((nn, _D), jnp.int32)
  return pl.pallas_call(
      body,
      grid=(nblk,),
      in_specs=[
          pl.BlockSpec((_BLK, _D), lambda i: (blk0 + i, 0)),
          pl.BlockSpec((_DEG, _BLK, _D), lambda i: (0, i, 0)),
          pl.BlockSpec((_DEG, _BLK, 1), lambda i: (0, i, 0)),
          pl.BlockSpec((2 * _D, 4 * _D), full),
          pl.BlockSpec((1, 4 * _D), full),
          pl.BlockSpec((2 * _D, _D), full),
          pl.BlockSpec((1, _D), full),
      ],
      out_specs=out_specs,
      out_shape=out_shape,
  )(hp, m, ew, w_gates, bias, w_sn, b_neigh)


def kernel(x, nbr1, nbr2, nbr3, ew1, ew2, ew3,
           Wih1, Whh1, bih1, bhh1, Wself1, Wneigh1, bneigh1,
           Wih2, Whh2, bih2, bhh2, Wself2, Wneigh2, bneigh2,
           Wih3, Whh3, bih3, bhh3, Wself3, Wneigh3, bneigh3):
  xb = x.astype(jnp.bfloat16)
  hp = lax.bitcast_convert_type(jnp.transpose(xb, (0, 2, 1)), jnp.int32)

  layers = []
  for nbr, ew, Wih, Whh, bih, bhh, Wself, Wneigh, bneigh in (
      (nbr1, ew1, Wih1, Whh1, bih1, bhh1, Wself1, Wneigh1, bneigh1),
      (nbr2, ew2, Wih2, Whh2, bih2, bhh2, Wself2, Wneigh2, bneigh2),
      (nbr3, ew3, Wih3, Whh3, bih3, bhh3, Wself3, Wneigh3, bneigh3)):
    idx = [jnp.transpose(nbr[n0:n0 + nn], (1, 0)).reshape(_DEG * nn)
           for n0, nn in _CHUNKS]
    ew_b = [jnp.transpose(ew[n0:n0 + nn], (1, 0)).reshape(_DEG, nn, 1)
            for n0, nn in _CHUNKS]
    gscale = jnp.concatenate(
        [jnp.full((2 * _D,), 0.5), jnp.ones((_D,)), jnp.full((_D,), 0.5)]
    ).astype(jnp.float32)[None, :]
    w_gates = (jnp.concatenate([Wih.T, Whh.T], axis=0)
               * gscale).astype(jnp.bfloat16)
    bias = ((bih + bhh)[None, :] * gscale)
    w_sn = jnp.concatenate([Wself.T, Wneigh.T], axis=0)
    layers.append((idx, ew_b, w_gates, bias, w_sn, bneigh[None, :]))

  for li, (idx, ew_b, w_gates, bias, w_sn, b_neigh) in enumerate(layers):
    final = li == 2
    outs = []
    for ci, (n0, nn) in enumerate(_CHUNKS):
      cc = _SC_C if (nn % (2 * _SC_GRP) == 0) else _SC_C // 2
      m = _sc_gather(hp, idx[ci], c=cc).reshape(_DEG, nn, _D)
      outs.append(_tc_layer(hp, m, ew_b[ci], w_gates, bias, w_sn, b_neigh,
                            n0, nn, final))
    if final:
      c0 = jnp.concatenate([o[0] for o in outs], axis=0)
      c1 = jnp.concatenate([o[1] for o in outs], axis=0)
    else:
      hp = jnp.concatenate(outs, axis=0)

  return jnp.stack([c0, c1], axis=1)


# speedup vs baseline: 1.0264x; 1.0264x over previous
import functools

import jax
import jax.numpy as jnp
from jax import lax
from jax.experimental import pallas as pl
from jax.experimental.pallas import tpu as pltpu
from jax.experimental.pallas import tpu_sc as plsc

_N = 10000
_DEG = 32
_D = 128
_NUM = 2
_G = _N * _DEG

_SC_C = 80
_SC_K = 5
_SC_GRP = _SC_C * _SC_K
_NW = 32

_BLK = 200

_CHUNKS = ((0, 4000), (4000, 4000), (8000, 2000))


def _sc_gather(table, idx, c=_SC_C, k=_SC_K):
  grp = c * k
  g_total = idx.shape[0]
  d = table.shape[1]
  per_w = g_total // _NW
  ngrp = per_w // grp
  assert per_w % grp == 0

  mesh = plsc.VectorSubcoreMesh(core_axis_name="c", subcore_axis_name="s")

  @functools.partial(
      pl.kernel,
      out_type=jax.ShapeDtypeStruct((g_total, d), jnp.int32),
      mesh=mesh,
      scratch_types=[
          pltpu.VMEM((grp,), jnp.int32),
          pltpu.VMEM((grp, d), jnp.int32),
          pltpu.SemaphoreType.DMA,
      ],
  )
  def gather_k(table_hbm, idx_hbm, out_hbm, idx_v, rows_v, gsem):
    wid = lax.axis_index("s") * 2 + lax.axis_index("c")
    base = wid * per_w

    def group(gi, carry):
      gbase = base + gi * grp
      pltpu.sync_copy(idx_hbm.at[pl.ds(gbase, grp)], idx_v)
      copies = []
      for j in range(k):
        copies.append(
            pltpu.async_copy(
                table_hbm.at[idx_v.at[pl.ds(j * c, c)]],
                rows_v.at[pl.ds(j * c, c)],
                gsem,
            ))
      for cp in copies:
        cp.wait()
      pltpu.sync_copy(rows_v, out_hbm.at[pl.ds(gbase, grp)])
      return carry

    lax.fori_loop(0, ngrp, group, 0)

  return gather_k(table, idx)


def _sigm(v):
  return 0.5 + 0.5 * jnp.tanh(0.5 * v)


def _unpack2(v32):
  lo = lax.bitcast_convert_type(lax.shift_left(v32, 16), jnp.float32)
  hi = lax.bitcast_convert_type(
      lax.bitwise_and(v32, jnp.int32(-65536)), jnp.float32)
  return lo, hi


def _pack2(f_lo, f_hi):
  def rne(f):
    u = lax.bitcast_convert_type(f, jnp.uint32)
    return u + jnp.uint32(0x7FFF) + (
        lax.shift_right_logical(u, jnp.uint32(16)) & jnp.uint32(1))
  lo = lax.shift_right_logical(rne(f_lo), jnp.uint32(16))
  hi = lax.bitwise_and(rne(f_hi), jnp.uint32(0xFFFF0000))
  return lax.bitcast_convert_type(lax.bitwise_or(lo, hi), jnp.int32)


def _tc_layer(hp, m, ew, w_gates, bias, w_sn, b_neigh, n0, nn, final):
  nblk = nn // _BLK
  blk0 = n0 // _BLK
  b2 = 2 * _BLK

  def body(hp_ref, m_ref, ew_ref, wg_ref, b_ref, wsn_ref, bn_ref, *out_refs):
    h0lo, h0hi = _unpack2(hp_ref[...])
    h0 = jnp.concatenate([h0lo, h0hi], axis=0)
    wg = wg_ref[...]
    b = b_ref[...]
    ht = jnp.zeros((b2, _D), jnp.float32)
    ct = jnp.zeros((b2, _D), jnp.float32)
    for t in range(_DEG):
      et = ew_ref[t]
      mlo, mhi = _unpack2(m_ref[t])
      mt = jnp.concatenate([mlo * et, mhi * et], axis=0)
      g = jnp.dot(jnp.concatenate([mt, ht], axis=1).astype(jnp.bfloat16),
                  wg, preferred_element_type=jnp.float32) + b
      ig = 0.5 + 0.5 * jnp.tanh(g[:, :_D])
      fg = 0.5 + 0.5 * jnp.tanh(g[:, _D:2 * _D])
      gg = jnp.tanh(g[:, 2 * _D:3 * _D])
      og = 0.5 + 0.5 * jnp.tanh(g[:, 3 * _D:])
      ct = fg * ct + ig * gg
      ht = og * jnp.tanh(ct)
    out = (jnp.dot(jnp.concatenate([h0, ht], axis=1), wsn_ref[...],
                   preferred_element_type=jnp.float32) + bn_ref[...])
    if final:
      out_refs[0][...] = out[:_BLK]
      out_refs[1][...] = out[_BLK:]
    else:
      out_refs[0][...] = _pack2(out[:_BLK], out[_BLK:])

  full = lambda i: (0, 0)
  if final:
    out_specs = [pl.BlockSpec((_BLK, _D), lambda i: (i, 0)),
                 pl.BlockSpec((_BLK, _D), lambda i: (i, 0))]
    out_shape = [jax.ShapeDtypeStruct((nn, _D), jnp.float32),
                 jax.ShapeDtypeStruct((nn, _D), jnp.float32)]
  else:
    out_specs = pl.BlockSpec((_BLK, _D), lambda i: (i, 0))
    out_shape = jax.ShapeDtypeStruct((nn, _D), jnp.int32)
  return pl.pallas_call(
      body,
      grid=(nblk,),
      in_specs=[
          pl.BlockSpec((_BLK, _D), lambda i: (blk0 + i, 0)),
          pl.BlockSpec((_DEG, _BLK, _D), lambda i: (0, i, 0)),
          pl.BlockSpec((_DEG, _BLK, 1), lambda i: (0, i, 0)),
          pl.BlockSpec((2 * _D, 4 * _D), full),
          pl.BlockSpec((1, 4 * _D), full),
          pl.BlockSpec((2 * _D, _D), full),
          pl.BlockSpec((1, _D), full),
      ],
      out_specs=out_specs,
      out_shape=out_shape,
  )(hp, m, ew, w_gates, bias, w_sn, b_neigh)


def kernel(x, nbr1, nbr2, nbr3, ew1, ew2, ew3,
           Wih1, Whh1, bih1, bhh1, Wself1, Wneigh1, bneigh1,
           Wih2, Whh2, bih2, bhh2, Wself2, Wneigh2, bneigh2,
           Wih3, Whh3, bih3, bhh3, Wself3, Wneigh3, bneigh3):
  xb = x.astype(jnp.bfloat16)
  hp = lax.bitcast_convert_type(jnp.transpose(xb, (0, 2, 1)), jnp.int32)

  layers = []
  for nbr, ew, Wih, Whh, bih, bhh, Wself, Wneigh, bneigh in (
      (nbr1, ew1, Wih1, Whh1, bih1, bhh1, Wself1, Wneigh1, bneigh1),
      (nbr2, ew2, Wih2, Whh2, bih2, bhh2, Wself2, Wneigh2, bneigh2),
      (nbr3, ew3, Wih3, Whh3, bih3, bhh3, Wself3, Wneigh3, bneigh3)):
    idx = [jnp.transpose(nbr[n0:n0 + nn], (1, 0)).reshape(_DEG * nn)
           for n0, nn in _CHUNKS]
    ew_b = [jnp.transpose(ew[n0:n0 + nn], (1, 0)).reshape(_DEG, nn, 1)
            for n0, nn in _CHUNKS]
    gscale = jnp.concatenate(
        [jnp.full((2 * _D,), 0.5), jnp.ones((_D,)), jnp.full((_D,), 0.5)]
    ).astype(jnp.float32)[None, :]
    w_gates = (jnp.concatenate([Wih.T, Whh.T], axis=0)
               * gscale).astype(jnp.bfloat16)
    bias = ((bih + bhh)[None, :] * gscale)
    w_sn = jnp.concatenate([Wself.T, Wneigh.T], axis=0)
    layers.append((idx, ew_b, w_gates, bias, w_sn, bneigh[None, :]))

  for li, (idx, ew_b, w_gates, bias, w_sn, b_neigh) in enumerate(layers):
    final = li == 2
    outs = []
    for ci, (n0, nn) in enumerate(_CHUNKS):
      m = _sc_gather(hp, idx[ci]).reshape(_DEG, nn, _D)
      outs.append(_tc_layer(hp, m, ew_b[ci], w_gates, bias, w_sn, b_neigh,
                            n0, nn, final))
    if final:
      c0 = jnp.concatenate([o[0] for o in outs], axis=0)
      c1 = jnp.concatenate([o[1] for o in outs], axis=0)
    else:
      hp = jnp.concatenate(outs, axis=0)

  return jnp.stack([c0, c1], axis=1)


# speedup vs baseline: 1.0749x; 1.0472x over previous
import functools

import jax
import jax.numpy as jnp
from jax import lax
from jax.experimental import pallas as pl
from jax.experimental.pallas import tpu as pltpu
from jax.experimental.pallas import tpu_sc as plsc

_N = 10000
_DEG = 32
_D = 128
_NUM = 2
_G = _N * _DEG

_SC_C = 80
_SC_K = 5
_SC_GRP = _SC_C * _SC_K
_NW = 32

_BLK = 400

_CHUNKS = ((0, 4000), (4000, 4000), (8000, 2000))


def _sc_gather(table, idx, c=_SC_C, k=_SC_K):
  grp = c * k
  g_total = idx.shape[0]
  d = table.shape[1]
  per_w = g_total // _NW
  ngrp = per_w // grp
  assert per_w % grp == 0

  mesh = plsc.VectorSubcoreMesh(core_axis_name="c", subcore_axis_name="s")

  @functools.partial(
      pl.kernel,
      out_type=jax.ShapeDtypeStruct((g_total, d), jnp.int32),
      mesh=mesh,
      scratch_types=[
          pltpu.VMEM((grp,), jnp.int32),
          pltpu.VMEM((grp, d), jnp.int32),
          pltpu.SemaphoreType.DMA,
      ],
  )
  def gather_k(table_hbm, idx_hbm, out_hbm, idx_v, rows_v, gsem):
    wid = lax.axis_index("s") * 2 + lax.axis_index("c")
    base = wid * per_w

    def group(gi, carry):
      gbase = base + gi * grp
      pltpu.sync_copy(idx_hbm.at[pl.ds(gbase, grp)], idx_v)
      copies = []
      for j in range(k):
        copies.append(
            pltpu.async_copy(
                table_hbm.at[idx_v.at[pl.ds(j * c, c)]],
                rows_v.at[pl.ds(j * c, c)],
                gsem,
            ))
      for cp in copies:
        cp.wait()
      pltpu.sync_copy(rows_v, out_hbm.at[pl.ds(gbase, grp)])
      return carry

    lax.fori_loop(0, ngrp, group, 0)

  return gather_k(table, idx)


def _sigm(v):
  return 0.5 + 0.5 * jnp.tanh(0.5 * v)


def _unpack2(v32):
  lo = lax.bitcast_convert_type(lax.shift_left(v32, 16), jnp.float32)
  hi = lax.bitcast_convert_type(
      lax.bitwise_and(v32, jnp.int32(-65536)), jnp.float32)
  return lo, hi


def _pack2(f_lo, f_hi):
  def rne(f):
    u = lax.bitcast_convert_type(f, jnp.uint32)
    return u + jnp.uint32(0x7FFF) + (
        lax.shift_right_logical(u, jnp.uint32(16)) & jnp.uint32(1))
  lo = lax.shift_right_logical(rne(f_lo), jnp.uint32(16))
  hi = lax.bitwise_and(rne(f_hi), jnp.uint32(0xFFFF0000))
  return lax.bitcast_convert_type(lax.bitwise_or(lo, hi), jnp.int32)


def _tc_layer(hp, m, ew, w_gates, bias, w_sn, b_neigh, n0, nn, final):
  nblk = nn // _BLK
  blk0 = n0 // _BLK
  b2 = 2 * _BLK

  def body(hp_ref, m_ref, ew_ref, wg_ref, b_ref, wsn_ref, bn_ref, *out_refs):
    h0lo, h0hi = _unpack2(hp_ref[...])
    h0 = jnp.concatenate([h0lo, h0hi], axis=0)
    wg = wg_ref[...]
    b = b_ref[...]
    ht = jnp.zeros((b2, _D), jnp.float32)
    ct = jnp.zeros((b2, _D), jnp.float32)
    for t in range(_DEG):
      et = ew_ref[t]
      mlo, mhi = _unpack2(m_ref[t])
      mt = jnp.concatenate([mlo * et, mhi * et], axis=0)
      g = jnp.dot(jnp.concatenate([mt, ht], axis=1).astype(jnp.bfloat16),
                  wg, preferred_element_type=jnp.float32) + b
      ig = 0.5 + 0.5 * jnp.tanh(g[:, :_D])
      fg = 0.5 + 0.5 * jnp.tanh(g[:, _D:2 * _D])
      gg = jnp.tanh(g[:, 2 * _D:3 * _D])
      og = 0.5 + 0.5 * jnp.tanh(g[:, 3 * _D:])
      ct = fg * ct + ig * gg
      ht = og * jnp.tanh(ct)
    out = (jnp.dot(jnp.concatenate([h0, ht], axis=1), wsn_ref[...],
                   preferred_element_type=jnp.float32) + bn_ref[...])
    if final:
      out_refs[0][...] = out[:_BLK]
      out_refs[1][...] = out[_BLK:]
    else:
      out_refs[0][...] = _pack2(out[:_BLK], out[_BLK:])

  full = lambda i: (0, 0)
  if final:
    out_specs = [pl.BlockSpec((_BLK, _D), lambda i: (i, 0)),
                 pl.BlockSpec((_BLK, _D), lambda i: (i, 0))]
    out_shape = [jax.ShapeDtypeStruct((nn, _D), jnp.float32),
                 jax.ShapeDtypeStruct((nn, _D), jnp.float32)]
  else:
    out_specs = pl.BlockSpec((_BLK, _D), lambda i: (i, 0))
    out_shape = jax.ShapeDtypeStruct((nn, _D), jnp.int32)
  return pl.pallas_call(
      body,
      grid=(nblk,),
      in_specs=[
          pl.BlockSpec((_BLK, _D), lambda i: (blk0 + i, 0)),
          pl.BlockSpec((_DEG, _BLK, _D), lambda i: (0, i, 0)),
          pl.BlockSpec((_DEG, _BLK, 1), lambda i: (0, i, 0)),
          pl.BlockSpec((2 * _D, 4 * _D), full),
          pl.BlockSpec((1, 4 * _D), full),
          pl.BlockSpec((2 * _D, _D), full),
          pl.BlockSpec((1, _D), full),
      ],
      out_specs=out_specs,
      out_shape=out_shape,
  )(hp, m, ew, w_gates, bias, w_sn, b_neigh)


def kernel(x, nbr1, nbr2, nbr3, ew1, ew2, ew3,
           Wih1, Whh1, bih1, bhh1, Wself1, Wneigh1, bneigh1,
           Wih2, Whh2, bih2, bhh2, Wself2, Wneigh2, bneigh2,
           Wih3, Whh3, bih3, bhh3, Wself3, Wneigh3, bneigh3):
  xb = x.astype(jnp.bfloat16)
  hp = lax.bitcast_convert_type(jnp.transpose(xb, (0, 2, 1)), jnp.int32)

  layers = []
  for nbr, ew, Wih, Whh, bih, bhh, Wself, Wneigh, bneigh in (
      (nbr1, ew1, Wih1, Whh1, bih1, bhh1, Wself1, Wneigh1, bneigh1),
      (nbr2, ew2, Wih2, Whh2, bih2, bhh2, Wself2, Wneigh2, bneigh2),
      (nbr3, ew3, Wih3, Whh3, bih3, bhh3, Wself3, Wneigh3, bneigh3)):
    idx = [jnp.transpose(nbr[n0:n0 + nn], (1, 0)).reshape(_DEG * nn)
           for n0, nn in _CHUNKS]
    ew_b = [jnp.transpose(ew[n0:n0 + nn], (1, 0)).reshape(_DEG, nn, 1)
            for n0, nn in _CHUNKS]
    gscale = jnp.concatenate(
        [jnp.full((2 * _D,), 0.5), jnp.ones((_D,)), jnp.full((_D,), 0.5)]
    ).astype(jnp.float32)[None, :]
    w_gates = (jnp.concatenate([Wih.T, Whh.T], axis=0)
               * gscale).astype(jnp.bfloat16)
    bias = ((bih + bhh)[None, :] * gscale)
    w_sn = jnp.concatenate([Wself.T, Wneigh.T], axis=0)
    layers.append((idx, ew_b, w_gates, bias, w_sn, bneigh[None, :]))

  for li, (idx, ew_b, w_gates, bias, w_sn, b_neigh) in enumerate(layers):
    final = li == 2
    outs = []
    for ci, (n0, nn) in enumerate(_CHUNKS):
      m = _sc_gather(hp, idx[ci]).reshape(_DEG, nn, _D)
      outs.append(_tc_layer(hp, m, ew_b[ci], w_gates, bias, w_sn, b_neigh,
                            n0, nn, final))
    if final:
      c0 = jnp.concatenate([o[0] for o in outs], axis=0)
      c1 = jnp.concatenate([o[1] for o in outs], axis=0)
    else:
      hp = jnp.concatenate(outs, axis=0)

  return jnp.stack([c0, c1], axis=1)


# speedup vs baseline: 1.0774x; 1.0023x over previous
import functools

import jax
import jax.numpy as jnp
from jax import lax
from jax.experimental import pallas as pl
from jax.experimental.pallas import tpu as pltpu
from jax.experimental.pallas import tpu_sc as plsc

_N = 10000
_DEG = 32
_D = 128
_NUM = 2
_G = _N * _DEG

_SC_C = 80
_SC_K = 5
_SC_GRP = _SC_C * _SC_K
_NW = 32

_BLK = 400

_CHUNKS = ((0, 10000),)


def _sc_gather(table, idx, c=_SC_C, k=_SC_K):
  grp = c * k
  g_total = idx.shape[0]
  d = table.shape[1]
  per_w = g_total // _NW
  ngrp = per_w // grp
  assert per_w % grp == 0

  mesh = plsc.VectorSubcoreMesh(core_axis_name="c", subcore_axis_name="s")

  @functools.partial(
      pl.kernel,
      out_type=jax.ShapeDtypeStruct((g_total, d), jnp.int32),
      mesh=mesh,
      scratch_types=[
          pltpu.VMEM((grp,), jnp.int32),
          pltpu.VMEM((grp, d), jnp.int32),
          pltpu.SemaphoreType.DMA,
      ],
  )
  def gather_k(table_hbm, idx_hbm, out_hbm, idx_v, rows_v, gsem):
    wid = lax.axis_index("s") * 2 + lax.axis_index("c")
    base = wid * per_w

    def group(gi, carry):
      gbase = base + gi * grp
      pltpu.sync_copy(idx_hbm.at[pl.ds(gbase, grp)], idx_v)
      copies = []
      for j in range(k):
        copies.append(
            pltpu.async_copy(
                table_hbm.at[idx_v.at[pl.ds(j * c, c)]],
                rows_v.at[pl.ds(j * c, c)],
                gsem,
            ))
      for cp in copies:
        cp.wait()
      pltpu.sync_copy(rows_v, out_hbm.at[pl.ds(gbase, grp)])
      return carry

    lax.fori_loop(0, ngrp, group, 0)

  return gather_k(table, idx)


def _sigm(v):
  return 0.5 + 0.5 * jnp.tanh(0.5 * v)


def _unpack2(v32):
  lo = lax.bitcast_convert_type(lax.shift_left(v32, 16), jnp.float32)
  hi = lax.bitcast_convert_type(
      lax.bitwise_and(v32, jnp.int32(-65536)), jnp.float32)
  return lo, hi


def _pack2(f_lo, f_hi):
  def rne(f):
    u = lax.bitcast_convert_type(f, jnp.uint32)
    return u + jnp.uint32(0x7FFF) + (
        lax.shift_right_logical(u, jnp.uint32(16)) & jnp.uint32(1))
  lo = lax.shift_right_logical(rne(f_lo), jnp.uint32(16))
  hi = lax.bitwise_and(rne(f_hi), jnp.uint32(0xFFFF0000))
  return lax.bitcast_convert_type(lax.bitwise_or(lo, hi), jnp.int32)


def _tc_layer(hp, m, ew, w_gates, bias, w_sn, b_neigh, n0, nn, final):
  nblk = nn // _BLK
  blk0 = n0 // _BLK
  b2 = 2 * _BLK

  def body(hp_ref, m_ref, ew_ref, wg_ref, b_ref, wsn_ref, bn_ref, *out_refs):
    h0lo, h0hi = _unpack2(hp_ref[...])
    h0 = jnp.concatenate([h0lo, h0hi], axis=0)
    wg = wg_ref[...]
    b = b_ref[...]
    ht = jnp.zeros((b2, _D), jnp.float32)
    ct = jnp.zeros((b2, _D), jnp.float32)
    for t in range(_DEG):
      et = ew_ref[t]
      mlo, mhi = _unpack2(m_ref[t])
      mt = jnp.concatenate([mlo * et, mhi * et], axis=0)
      g = jnp.dot(jnp.concatenate([mt, ht], axis=1).astype(jnp.bfloat16),
                  wg, preferred_element_type=jnp.float32) + b
      ig = 0.5 + 0.5 * jnp.tanh(g[:, :_D])
      fg = 0.5 + 0.5 * jnp.tanh(g[:, _D:2 * _D])
      gg = jnp.tanh(g[:, 2 * _D:3 * _D])
      og = 0.5 + 0.5 * jnp.tanh(g[:, 3 * _D:])
      ct = fg * ct + ig * gg
      ht = og * jnp.tanh(ct)
    out = (jnp.dot(jnp.concatenate([h0, ht], axis=1), wsn_ref[...],
                   preferred_element_type=jnp.float32) + bn_ref[...])
    if final:
      out_refs[0][...] = out[:_BLK]
      out_refs[1][...] = out[_BLK:]
    else:
      out_refs[0][...] = _pack2(out[:_BLK], out[_BLK:])

  full = lambda i: (0, 0)
  if final:
    out_specs = [pl.BlockSpec((_BLK, _D), lambda i: (i, 0)),
                 pl.BlockSpec((_BLK, _D), lambda i: (i, 0))]
    out_shape = [jax.ShapeDtypeStruct((nn, _D), jnp.float32),
                 jax.ShapeDtypeStruct((nn, _D), jnp.float32)]
  else:
    out_specs = pl.BlockSpec((_BLK, _D), lambda i: (i, 0))
    out_shape = jax.ShapeDtypeStruct((nn, _D), jnp.int32)
  return pl.pallas_call(
      body,
      grid=(nblk,),
      in_specs=[
          pl.BlockSpec((_BLK, _D), lambda i: (blk0 + i, 0)),
          pl.BlockSpec((_DEG, _BLK, _D), lambda i: (0, i, 0)),
          pl.BlockSpec((_DEG, _BLK, 1), lambda i: (0, i, 0)),
          pl.BlockSpec((2 * _D, 4 * _D), full),
          pl.BlockSpec((1, 4 * _D), full),
          pl.BlockSpec((2 * _D, _D), full),
          pl.BlockSpec((1, _D), full),
      ],
      out_specs=out_specs,
      out_shape=out_shape,
  )(hp, m, ew, w_gates, bias, w_sn, b_neigh)


def kernel(x, nbr1, nbr2, nbr3, ew1, ew2, ew3,
           Wih1, Whh1, bih1, bhh1, Wself1, Wneigh1, bneigh1,
           Wih2, Whh2, bih2, bhh2, Wself2, Wneigh2, bneigh2,
           Wih3, Whh3, bih3, bhh3, Wself3, Wneigh3, bneigh3):
  xb = x.astype(jnp.bfloat16)
  hp = lax.bitcast_convert_type(jnp.transpose(xb, (0, 2, 1)), jnp.int32)

  layers = []
  for nbr, ew, Wih, Whh, bih, bhh, Wself, Wneigh, bneigh in (
      (nbr1, ew1, Wih1, Whh1, bih1, bhh1, Wself1, Wneigh1, bneigh1),
      (nbr2, ew2, Wih2, Whh2, bih2, bhh2, Wself2, Wneigh2, bneigh2),
      (nbr3, ew3, Wih3, Whh3, bih3, bhh3, Wself3, Wneigh3, bneigh3)):
    idx = [jnp.transpose(nbr[n0:n0 + nn], (1, 0)).reshape(_DEG * nn)
           for n0, nn in _CHUNKS]
    ew_b = [jnp.transpose(ew[n0:n0 + nn], (1, 0)).reshape(_DEG, nn, 1)
            for n0, nn in _CHUNKS]
    gscale = jnp.concatenate(
        [jnp.full((2 * _D,), 0.5), jnp.ones((_D,)), jnp.full((_D,), 0.5)]
    ).astype(jnp.float32)[None, :]
    w_gates = (jnp.concatenate([Wih.T, Whh.T], axis=0)
               * gscale).astype(jnp.bfloat16)
    bias = ((bih + bhh)[None, :] * gscale)
    w_sn = jnp.concatenate([Wself.T, Wneigh.T], axis=0)
    layers.append((idx, ew_b, w_gates, bias, w_sn, bneigh[None, :]))

  for li, (idx, ew_b, w_gates, bias, w_sn, b_neigh) in enumerate(layers):
    final = li == 2
    outs = []
    for ci, (n0, nn) in enumerate(_CHUNKS):
      m = _sc_gather(hp, idx[ci]).reshape(_DEG, nn, _D)
      outs.append(_tc_layer(hp, m, ew_b[ci], w_gates, bias, w_sn, b_neigh,
                            n0, nn, final))
    if final:
      c0 = jnp.concatenate([o[0] for o in outs], axis=0)
      c1 = jnp.concatenate([o[1] for o in outs], axis=0)
    else:
      hp = jnp.concatenate(outs, axis=0)

  return jnp.stack([c0, c1], axis=1)


# speedup vs baseline: 1.0845x; 1.0065x over previous
import functools

import jax
import jax.numpy as jnp
from jax import lax
from jax.experimental import pallas as pl
from jax.experimental.pallas import tpu as pltpu
from jax.experimental.pallas import tpu_sc as plsc

_N = 10000
_DEG = 32
_D = 128
_NUM = 2
_G = _N * _DEG

_SC_C = 80
_SC_K = 5
_SC_GRP = _SC_C * _SC_K
_NW = 32

_BLK = 400

_CHUNKS = ((0, 10000),)


def _sc_gather(table, idx, c=_SC_C, k=_SC_K):
  grp = c * k
  g_total = idx.shape[0]
  d = table.shape[1]
  per_w = g_total // _NW
  ngrp = per_w // grp
  assert per_w % grp == 0

  mesh = plsc.VectorSubcoreMesh(core_axis_name="c", subcore_axis_name="s")

  @functools.partial(
      pl.kernel,
      out_type=jax.ShapeDtypeStruct((g_total, d), jnp.int32),
      mesh=mesh,
      scratch_types=[
          pltpu.VMEM((grp,), jnp.int32),
          pltpu.VMEM((grp, d), jnp.int32),
          pltpu.SemaphoreType.DMA,
      ],
  )
  def gather_k(table_hbm, idx_hbm, out_hbm, idx_v, rows_v, gsem):
    wid = lax.axis_index("s") * 2 + lax.axis_index("c")
    base = wid * per_w

    def group(gi, carry):
      gbase = base + gi * grp
      pltpu.sync_copy(idx_hbm.at[pl.ds(gbase, grp)], idx_v)
      copies = []
      for j in range(k):
        copies.append(
            pltpu.async_copy(
                table_hbm.at[idx_v.at[pl.ds(j * c, c)]],
                rows_v.at[pl.ds(j * c, c)],
                gsem,
            ))
      for cp in copies:
        cp.wait()
      pltpu.sync_copy(rows_v, out_hbm.at[pl.ds(gbase, grp)])
      return carry

    lax.fori_loop(0, ngrp, group, 0)

  return gather_k(table, idx)


def _sigm(v):
  return 0.5 + 0.5 * jnp.tanh(0.5 * v)


def _unpack2(v32):
  lo = lax.bitcast_convert_type(lax.shift_left(v32, 16), jnp.float32)
  hi = lax.bitcast_convert_type(
      lax.bitwise_and(v32, jnp.int32(-65536)), jnp.float32)
  return lo, hi


def _pack2(f_lo, f_hi):
  def rne(f):
    u = lax.bitcast_convert_type(f, jnp.uint32)
    return u + jnp.uint32(0x7FFF) + (
        lax.shift_right_logical(u, jnp.uint32(16)) & jnp.uint32(1))
  lo = lax.shift_right_logical(rne(f_lo), jnp.uint32(16))
  hi = lax.bitwise_and(rne(f_hi), jnp.uint32(0xFFFF0000))
  return lax.bitcast_convert_type(lax.bitwise_or(lo, hi), jnp.int32)


def _tc_layer(hp, m, ew, w_gates, bias, w_sn, b_neigh, n0, nn, final):
  nblk = nn // _BLK
  blk0 = n0 // _BLK
  b2 = 2 * _BLK

  def body(hp_ref, m_ref, ew_ref, wg_ref, b_ref, wsn_ref, bn_ref, *out_refs):
    h0lo, h0hi = _unpack2(hp_ref[...])
    h0 = jnp.concatenate([h0lo, h0hi], axis=0)
    wg = wg_ref[...]
    b = b_ref[...]
    ht = jnp.zeros((b2, _D), jnp.float32)
    ct = jnp.zeros((b2, _D), jnp.float32)
    for t in range(_DEG):
      et = ew_ref[t].astype(jnp.float32)
      mlo, mhi = _unpack2(m_ref[t])
      mt = jnp.concatenate([mlo * et, mhi * et], axis=0)
      g = jnp.dot(jnp.concatenate([mt, ht], axis=1).astype(jnp.bfloat16),
                  wg, preferred_element_type=jnp.float32) + b
      ig = 0.5 + 0.5 * jnp.tanh(g[:, :_D])
      fg = 0.5 + 0.5 * jnp.tanh(g[:, _D:2 * _D])
      gg = jnp.tanh(g[:, 2 * _D:3 * _D])
      og = 0.5 + 0.5 * jnp.tanh(g[:, 3 * _D:])
      ct = fg * ct + ig * gg
      ht = og * jnp.tanh(ct)
    out = (jnp.dot(jnp.concatenate([h0, ht], axis=1), wsn_ref[...],
                   preferred_element_type=jnp.float32) + bn_ref[...])
    if final:
      out_refs[0][...] = out[:_BLK]
      out_refs[1][...] = out[_BLK:]
    else:
      out_refs[0][...] = _pack2(out[:_BLK], out[_BLK:])

  full = lambda i: (0, 0)
  if final:
    out_specs = [pl.BlockSpec((_BLK, _D), lambda i: (i, 0)),
                 pl.BlockSpec((_BLK, _D), lambda i: (i, 0))]
    out_shape = [jax.ShapeDtypeStruct((nn, _D), jnp.float32),
                 jax.ShapeDtypeStruct((nn, _D), jnp.float32)]
  else:
    out_specs = pl.BlockSpec((_BLK, _D), lambda i: (i, 0))
    out_shape = jax.ShapeDtypeStruct((nn, _D), jnp.int32)
  return pl.pallas_call(
      body,
      grid=(nblk,),
      in_specs=[
          pl.BlockSpec((_BLK, _D), lambda i: (blk0 + i, 0)),
          pl.BlockSpec((_DEG, _BLK, _D), lambda i: (0, i, 0)),
          pl.BlockSpec((_DEG, _BLK, 1), lambda i: (0, i, 0)),
          pl.BlockSpec((2 * _D, 4 * _D), full),
          pl.BlockSpec((1, 4 * _D), full),
          pl.BlockSpec((2 * _D, _D), full),
          pl.BlockSpec((1, _D), full),
      ],
      out_specs=out_specs,
      out_shape=out_shape,
  )(hp, m, ew, w_gates, bias, w_sn, b_neigh)


def kernel(x, nbr1, nbr2, nbr3, ew1, ew2, ew3,
           Wih1, Whh1, bih1, bhh1, Wself1, Wneigh1, bneigh1,
           Wih2, Whh2, bih2, bhh2, Wself2, Wneigh2, bneigh2,
           Wih3, Whh3, bih3, bhh3, Wself3, Wneigh3, bneigh3):
  xb = x.astype(jnp.bfloat16)
  hp = lax.bitcast_convert_type(jnp.transpose(xb, (0, 2, 1)), jnp.int32)

  layers = []
  for nbr, ew, Wih, Whh, bih, bhh, Wself, Wneigh, bneigh in (
      (nbr1, ew1, Wih1, Whh1, bih1, bhh1, Wself1, Wneigh1, bneigh1),
      (nbr2, ew2, Wih2, Whh2, bih2, bhh2, Wself2, Wneigh2, bneigh2),
      (nbr3, ew3, Wih3, Whh3, bih3, bhh3, Wself3, Wneigh3, bneigh3)):
    idx = [jnp.transpose(nbr[n0:n0 + nn], (1, 0)).reshape(_DEG * nn)
           for n0, nn in _CHUNKS]
    ew_b = [jnp.transpose(ew[n0:n0 + nn], (1, 0)).reshape(_DEG, nn, 1)
            .astype(jnp.bfloat16) for n0, nn in _CHUNKS]
    gscale = jnp.concatenate(
        [jnp.full((2 * _D,), 0.5), jnp.ones((_D,)), jnp.full((_D,), 0.5)]
    ).astype(jnp.float32)[None, :]
    w_gates = (jnp.concatenate([Wih.T, Whh.T], axis=0)
               * gscale).astype(jnp.bfloat16)
    bias = ((bih + bhh)[None, :] * gscale)
    w_sn = jnp.concatenate([Wself.T, Wneigh.T], axis=0)
    layers.append((idx, ew_b, w_gates, bias, w_sn, bneigh[None, :]))

  for li, (idx, ew_b, w_gates, bias, w_sn, b_neigh) in enumerate(layers):
    final = li == 2
    outs = []
    for ci, (n0, nn) in enumerate(_CHUNKS):
      m = _sc_gather(hp, idx[ci]).reshape(_DEG, nn, _D)
      outs.append(_tc_layer(hp, m, ew_b[ci], w_gates, bias, w_sn, b_neigh,
                            n0, nn, final))
    if final:
      c0 = jnp.concatenate([o[0] for o in outs], axis=0)
      c1 = jnp.concatenate([o[1] for o in outs], axis=0)
    else:
      hp = jnp.concatenate(outs, axis=0)

  return jnp.stack([c0, c1], axis=1)
